# Initial kernel scaffold; baseline (speedup 1.0000x reference)
import jax
import jax.numpy as jnp
from jax.experimental import pallas as pl


def kernel(x, scatter, w1, s1, b1, wd, sd, bd, w2, s2, b2):
    raise NotImplementedError("write your pallas kernel here")


# speedup vs baseline: 2.3693x; 2.3693x over previous
import jax
import jax.numpy as jnp
from jax import lax
from jax.experimental import pallas as pl
from jax.experimental.pallas import tpu as pltpu


def _block_kernel(x_ref, w1_ref, s1_ref, b1_ref, wd_ref, sd_ref, bd_ref,
                  w2_ref, s2_ref, b2_ref, o_ref, pad_ref, *, c1):
    _, D, H, W, C = x_ref.shape
    M = D * H * W
    ch = C - c1
    Cd = 2 * ch
    xa = x_ref[0]
    xb = x_ref[1]
    x2 = jnp.concatenate([xa[..., c1:], xb[..., c1:]], axis=-1).reshape(M, Cd)
    xp = jnp.concatenate([xa[..., :c1], xb[..., :c1]], axis=-1).reshape(M, 2 * c1)

    y = jnp.dot(x2.astype(jnp.bfloat16), w1_ref[...],
                preferred_element_type=jnp.float32)
    y = jnp.maximum(y * s1_ref[...] + b1_ref[...], 0.0)

    pad_ref[0:1] = jnp.zeros((1, H + 2, W + 2, Cd), jnp.float32)
    pad_ref[D + 1:D + 2] = jnp.zeros((1, H + 2, W + 2, Cd), jnp.float32)
    pad_ref[:, 0:1] = jnp.zeros((D + 2, 1, W + 2, Cd), jnp.float32)
    pad_ref[:, H + 1:H + 2] = jnp.zeros((D + 2, 1, W + 2, Cd), jnp.float32)
    pad_ref[:, :, 0:1] = jnp.zeros((D + 2, H + 2, 1, Cd), jnp.float32)
    pad_ref[:, :, W + 1:W + 2] = jnp.zeros((D + 2, H + 2, 1, Cd), jnp.float32)
    pad_ref[1:D + 1, 1:H + 1, 1:W + 1, :] = y.reshape(D, H, W, Cd)
    acc = jnp.zeros((D, H, W, Cd), jnp.float32)
    for kd in range(3):
        for kh in range(3):
            for kw in range(3):
                idx = kd * 9 + kh * 3 + kw
                tap = pad_ref[kd:kd + D, kh:kh + H, kw:kw + W, :]
                acc = acc + tap * wd_ref[idx:idx + 1, :]
    z = acc * sd_ref[...] + bd_ref[...]

    u = jnp.concatenate([z.reshape(M, Cd).astype(jnp.bfloat16),
                         xp.astype(jnp.bfloat16)], axis=-1)
    v = jnp.dot(u, w2_ref[...], preferred_element_type=jnp.float32)
    g = v * s2_ref[...] + b2_ref[...]
    odd = (lax.broadcasted_iota(jnp.int32, (1, g.shape[1]), 1) % 2) == 1
    out = jnp.where(odd, jnp.maximum(g, 0.0), g)
    o_ref[0] = out[:, :C].astype(o_ref.dtype)
    o_ref[1] = out[:, C:].astype(o_ref.dtype)


def _bcast_spec(a):
    return pl.BlockSpec(a.shape, lambda n: (0,) * a.ndim)


def kernel(x, scatter, w1, s1, b1, wd, sd, bd, w2, s2, b2):
    N, C, D, H, W = x.shape
    c1 = scatter.shape[0]
    cm = w1.shape[1]
    oup = w2.shape[1]
    M = D * H * W
    bf = jnp.bfloat16

    xt = jnp.transpose(x, (0, 2, 3, 4, 1))

    w1b = jnp.zeros((2 * (C - c1), 2 * cm), jnp.float32)
    w1b = w1b.at[:C - c1, :cm].set(w1).at[C - c1:, cm:].set(w1).astype(bf)
    s1p = jnp.tile(s1, (1, 2))
    b1p = jnp.tile(b1, (1, 2))
    wdp = jnp.tile(wd, (1, 2))
    sdp = jnp.tile(sd, (1, 2))
    bdp = jnp.tile(bd, (1, 2))
    Wc = jnp.zeros((2 * cm + 2 * c1, 2 * oup), jnp.float32)
    Wc = Wc.at[:cm, :oup].set(w2).at[cm:2 * cm, oup:].set(w2)
    Wc = Wc.at[2 * cm:2 * cm + c1, :oup].set(scatter)
    Wc = Wc.at[2 * cm + c1:, oup:].set(scatter).astype(bf)
    even = (jnp.arange(oup) % 2 == 0).astype(jnp.float32)[None, :]
    s2c = jnp.tile(s2 + even, (1, 2))
    b2c = jnp.tile(b2, (1, 2))

    args = (xt, w1b, s1p, b1p, wdp, sdp, bdp, Wc, s2c, b2c)
    in_specs = [pl.BlockSpec((2, D, H, W, C), lambda n: (n, 0, 0, 0, 0))]
    in_specs += [_bcast_spec(a) for a in args[1:]]
    import functools
    out = pl.pallas_call(
        functools.partial(_block_kernel, c1=c1),
        out_shape=jax.ShapeDtypeStruct((N, M, oup), x.dtype),
        grid=(N // 2,),
        in_specs=in_specs,
        out_specs=pl.BlockSpec((2, M, oup), lambda n: (n, 0, 0)),
        scratch_shapes=[pltpu.VMEM((D + 2, H + 2, W + 2, 2 * cm), jnp.float32)],
        compiler_params=pltpu.CompilerParams(dimension_semantics=("parallel",)),
    )(*args)
    out = out.reshape(N, D, H, W, oup)
    return jnp.transpose(out, (0, 4, 1, 2, 3))
